# baseline (speedup 1.0000x reference)
import functools

import jax
import jax.numpy as jnp
from jax import lax
from jax.experimental import pallas as pl
from jax.experimental.pallas import tpu as pltpu
from jax.experimental.pallas import tpu_sc as plsc

B, K, D = 8192, 100, 100
D_PAD = 128
BH = B // 2
BLK = 2048

_NC, _NS = 2, 16
_NW = _NC * _NS
_CHUNK = 128
_NCHUNK = BH // _NW // _CHUNK


def _scores_t(x_ref, e):
    en = jnp.sum(e * e, axis=1, keepdims=True)
    scores_t = lax.dot_general(e, x_ref[...], (((1,), (1,)), ((), ())),
                               preferred_element_type=jnp.float32,
                               precision=lax.Precision.HIGHEST)
    return en - 2.0 * scores_t


def _argmin_body(x_ref, e_ref, idx_ref, ep_ref):
    e = e_ref[...]
    total = _scores_t(x_ref, e)
    idx_ref[...] = jnp.argmin(total, axis=0).astype(jnp.int32)

    @pl.when(pl.program_id(0) == 0)
    def _():
        ep_ref[:, :D] = e
        ep_ref[:, D:] = jnp.zeros((K, D_PAD - D), jnp.float32)


def _gather_body(x_ref, e_ref, out_ref):
    e = e_ref[...]
    total = _scores_t(x_ref, e)
    idx = jnp.argmin(total, axis=0)
    onehot_t = (lax.broadcasted_iota(jnp.int32, (K, BLK), 0)
                == idx[None, :]).astype(jnp.float32)
    out_ref[...] = lax.dot_general(onehot_t, e, (((0,), (0,)), ((), ())),
                                   preferred_element_type=jnp.float32,
                                   precision=lax.Precision.HIGHEST)


def _tc_argmin(x, e, interpret=False):
    return pl.pallas_call(
        _argmin_body,
        grid=(BH // BLK,),
        in_specs=[
            pl.BlockSpec((BLK, D), lambda i: (i, 0)),
            pl.BlockSpec((K, D), lambda i: (0, 0)),
        ],
        out_specs=[
            pl.BlockSpec((BLK,), lambda i: (i,)),
            pl.BlockSpec((K, D_PAD), lambda i: (0, 0)),
        ],
        out_shape=[
            jax.ShapeDtypeStruct((BH,), jnp.int32),
            jax.ShapeDtypeStruct((K, D_PAD), jnp.float32),
        ],
        interpret=interpret,
    )(x, e)


def _tc_gather(x, e, interpret=False):
    nh = BH // BLK
    return pl.pallas_call(
        _gather_body,
        grid=(nh,),
        in_specs=[
            pl.BlockSpec((BLK, D), lambda i: (nh + i, 0)),
            pl.BlockSpec((K, D), lambda i: (0, 0)),
        ],
        out_specs=pl.BlockSpec((BLK, D), lambda i: (i, 0)),
        out_shape=jax.ShapeDtypeStruct((BH, D), jnp.float32),
        interpret=interpret,
    )(x, e)


def _sc_gather_body(table_hbm, idx_hbm, out_hbm, idx_v, rows_v, *sems):
    gsems, wsem = sems[:_NCHUNK], sems[_NCHUNK]
    wid = lax.axis_index("s") * _NC + lax.axis_index("c")
    row0 = wid * _NCHUNK
    pltpu.sync_copy(idx_hbm.at[pl.ds(row0, _NCHUNK)], idx_v)
    gathers = [
        pltpu.async_copy(table_hbm.at[idx_v.at[j]], rows_v.at[j], gsems[j])
        for j in range(_NCHUNK)
    ]
    writes = []
    for j in range(_NCHUNK):
        gathers[j].wait()
        writes.append(
            pltpu.async_copy(rows_v.at[j], out_hbm.at[row0 + j], wsem))
    for w in writes:
        w.wait()


@functools.cache
def _sc_gather():
    return pl.kernel(
        _sc_gather_body,
        mesh=plsc.VectorSubcoreMesh(core_axis_name="c", subcore_axis_name="s"),
        out_type=jax.ShapeDtypeStruct((_NW * _NCHUNK, _CHUNK, D_PAD),
                                      jnp.float32),
        scratch_types=[
            pltpu.VMEM((_NCHUNK, _CHUNK), jnp.int32),
            pltpu.VMEM((_NCHUNK, _CHUNK, D_PAD), jnp.float32),
        ] + [pltpu.SemaphoreType.DMA] * (_NCHUNK + 1),
    )


def _assemble_body(a_ref, b_ref, out_ref):
    out_ref[:BH, :] = a_ref[:, :D]
    out_ref[BH:, :] = b_ref[...]


def _assemble(a, b, interpret=False):
    return pl.pallas_call(
        _assemble_body,
        out_shape=jax.ShapeDtypeStruct((B, D), jnp.float32),
        interpret=interpret,
    )(a, b)


def kernel(inputs, embeddings):
    idx0, ep = _tc_argmin(inputs, embeddings)
    out0_pad = _sc_gather()(ep, idx0.reshape(_NW * _NCHUNK, _CHUNK))
    out1 = _tc_gather(inputs, embeddings)
    return _assemble(out0_pad.reshape(BH, D_PAD), out1)


# speedup vs baseline: 1.0987x; 1.0987x over previous
import functools

import jax
import jax.numpy as jnp
from jax import lax
from jax.experimental import pallas as pl
from jax.experimental.pallas import tpu as pltpu
from jax.experimental.pallas import tpu_sc as plsc

B, K, D = 8192, 100, 100
D_PAD = 128
BH = B // 2
BLK = 2048

_NC, _NS = 2, 16
_NW = _NC * _NS
_CHUNK = 128
_NCHUNK = BH // _NW // _CHUNK


def _scores_t(x_ref, e):
    en = jnp.sum(e * e, axis=1, keepdims=True)
    scores_t = lax.dot_general(e, x_ref[...], (((1,), (1,)), ((), ())),
                               preferred_element_type=jnp.float32,
                               precision=lax.Precision.HIGHEST)
    return en - 2.0 * scores_t


def _argmin_body(x_ref, e_ref, idx_ref, ep_ref):
    e = e_ref[...]
    total = _scores_t(x_ref, e)
    idx_ref[...] = jnp.argmin(total, axis=0).astype(jnp.int32)

    @pl.when(pl.program_id(0) == 0)
    def _():
        ep_ref[:, :D] = e
        ep_ref[:, D:] = jnp.zeros((K, D_PAD - D), jnp.float32)


def _gather_body(x_ref, e_ref, out_ref):
    e = e_ref[...]
    total = _scores_t(x_ref, e)
    idx = jnp.argmin(total, axis=0)
    onehot_t = (lax.broadcasted_iota(jnp.int32, (K, BLK), 0)
                == idx[None, :]).astype(jnp.float32)
    out_ref[...] = lax.dot_general(onehot_t, e, (((0,), (0,)), ((), ())),
                                   preferred_element_type=jnp.float32,
                                   precision=lax.Precision.HIGHEST)


def _tc_argmin(x, e, interpret=False):
    return pl.pallas_call(
        _argmin_body,
        grid=(BH // BLK,),
        in_specs=[
            pl.BlockSpec((BLK, D), lambda i: (i, 0)),
            pl.BlockSpec((K, D), lambda i: (0, 0)),
        ],
        out_specs=[
            pl.BlockSpec((BLK,), lambda i: (i,)),
            pl.BlockSpec((K, D_PAD), lambda i: (0, 0)),
        ],
        out_shape=[
            jax.ShapeDtypeStruct((BH,), jnp.int32),
            jax.ShapeDtypeStruct((K, D_PAD), jnp.float32),
        ],
        interpret=interpret,
    )(x, e)


def _tc_gather(x, e, interpret=False):
    nh = BH // BLK
    return pl.pallas_call(
        _gather_body,
        grid=(nh,),
        in_specs=[
            pl.BlockSpec((BLK, D), lambda i: (nh + i, 0)),
            pl.BlockSpec((K, D), lambda i: (0, 0)),
        ],
        out_specs=pl.BlockSpec((BLK, D), lambda i: (i, 0)),
        out_shape=jax.ShapeDtypeStruct((BH, D), jnp.float32),
        interpret=interpret,
    )(x, e)


def _sc_gather_body(table_hbm, idx_hbm, out_hbm, idx_v, rows_v, *sems):
    gsems, wsem = sems[:_NCHUNK], sems[_NCHUNK]
    wid = lax.axis_index("s") * _NC + lax.axis_index("c")
    row0 = wid * _NCHUNK
    pltpu.sync_copy(idx_hbm.at[pl.ds(row0, _NCHUNK)], idx_v)
    gathers = [
        pltpu.async_copy(table_hbm.at[idx_v.at[j]], rows_v.at[j], gsems[j])
        for j in range(_NCHUNK)
    ]
    writes = []
    for j in range(_NCHUNK):
        gathers[j].wait()
        writes.append(
            pltpu.async_copy(rows_v.at[j], out_hbm.at[row0 + j], wsem))
    for w in writes:
        w.wait()


@functools.cache
def _sc_gather():
    return pl.kernel(
        _sc_gather_body,
        mesh=plsc.VectorSubcoreMesh(core_axis_name="c", subcore_axis_name="s"),
        out_type=jax.ShapeDtypeStruct((_NW * _NCHUNK, _CHUNK, D_PAD),
                                      jnp.float32),
        scratch_types=[
            pltpu.VMEM((_NCHUNK, _CHUNK), jnp.int32),
            pltpu.VMEM((_NCHUNK, _CHUNK, D_PAD), jnp.float32),
        ] + [pltpu.SemaphoreType.DMA] * (_NCHUNK + 1),
    )


def kernel(inputs, embeddings):
    idx0, ep = _tc_argmin(inputs, embeddings)
    out0_pad = _sc_gather()(ep, idx0.reshape(_NW * _NCHUNK, _CHUNK))
    out1 = _tc_gather(inputs, embeddings)
    out0 = out0_pad.reshape(BH, D_PAD)[:, :D]
    return jnp.concatenate([out0, out1], axis=0)
